# baseline (speedup 1.0000x reference)
import functools

import jax
import jax.numpy as jnp
from jax.experimental import pallas as pl
from jax.experimental.pallas import tpu as pltpu


def _msgs_from_h(h_blk, wef_ref, bef_ref, msgs_out_ref, nt, d):
    m = (jnp.dot(h_blk, wef_ref[:], preferred_element_type=jnp.float32)
         + bef_ref[:])
    for t in range(nt):
        msgs_out_ref[t] = m[:, t * d:(t + 1) * d].astype(jnp.bfloat16)


def _gather_kernel(labels_ref, emb_ref, wef_ref, bef_ref,
                   h_ref, msgs_ref, *, nt, d):
    n = labels_ref.shape[0]
    lpad = emb_ref.shape[0]
    lab = labels_ref[:]
    iota = jax.lax.broadcasted_iota(jnp.int32, (n, lpad), 1)
    onehot = (lab == iota).astype(jnp.float32)
    h0 = jnp.dot(onehot, emb_ref[:], preferred_element_type=jnp.float32)
    h_ref[:] = h0
    _msgs_from_h(h0, wef_ref, bef_ref, msgs_ref, nt, d)


def _gather_and_msgs(node_labels, emb, We_flat, be_flat, nt):
    n = node_labels.shape[0]
    l, d = emb.shape
    lpad = ((l + 127) // 128) * 128
    emb_p = jnp.pad(emb, ((0, lpad - l), (0, 0)))
    labels2d = node_labels.astype(jnp.int32).reshape(n, 1)
    return pl.pallas_call(
        functools.partial(_gather_kernel, nt=nt, d=d),
        out_shape=[
            jax.ShapeDtypeStruct((n, d), jnp.float32),
            jax.ShapeDtypeStruct((nt, n, d), jnp.bfloat16),
        ],
    )(labels2d, emb_p, We_flat, be_flat)


def _step_kernel(adj_ref, msgs_ref, h_ref, Wz_ref, Wr_ref, Wh_ref,
                 bz_ref, br_ref, bh_ref, wef_ref, bef_ref, *rest,
                 t_fwd, write_bf16):
    if write_bf16:
        out_ref, msgs_out_ref, adj16_ref = rest
    else:
        out_ref, msgs_out_ref = rest
        adj16_ref = None
    nt = msgs_ref.shape[0]
    d = msgs_ref.shape[2]

    slabs = []
    for t in range(nt):
        a = adj_ref[t]
        if a.dtype != jnp.bfloat16:
            a = a.astype(jnp.bfloat16)
        slabs.append(a)
        if write_bf16:
            adj16_ref[t] = a

    a_in = jnp.dot(slabs[0], msgs_ref[0], preferred_element_type=jnp.float32)
    for t in range(1, t_fwd):
        a_in += jnp.dot(slabs[t], msgs_ref[t], preferred_element_type=jnp.float32)
    a_out = jnp.dot(slabs[t_fwd], msgs_ref[t_fwd], preferred_element_type=jnp.float32)
    for t in range(t_fwd + 1, nt):
        a_out += jnp.dot(slabs[t], msgs_ref[t], preferred_element_type=jnp.float32)

    h_blk = h_ref[:]

    def lin3(a, b, c, w_ref, bias_ref):
        return (
            jnp.dot(a, w_ref[0:d, :], preferred_element_type=jnp.float32)
            + jnp.dot(b, w_ref[d:2 * d, :], preferred_element_type=jnp.float32)
            + jnp.dot(c, w_ref[2 * d:3 * d, :], preferred_element_type=jnp.float32)
            + bias_ref[:]
        )

    z = jax.nn.sigmoid(lin3(a_in, a_out, h_blk, Wz_ref, bz_ref))
    r = jax.nn.sigmoid(lin3(a_in, a_out, h_blk, Wr_ref, br_ref))
    h_hat = jnp.tanh(lin3(a_in, a_out, r * h_blk, Wh_ref, bh_ref))
    h_new = (1.0 - z) * h_blk + z * h_hat
    out_ref[:] = h_new
    _msgs_from_h(h_new, wef_ref, bef_ref, msgs_out_ref, nt, d)


def _prop_step(adj_tensor, msgs, h, Wz, bz, Wr, br, Wh, bh,
               We_flat, be_flat, *, bm, write_bf16):
    nt, n, _ = adj_tensor.shape
    d = h.shape[1]
    nb = n // bm
    t_fwd = nt // 2

    full = lambda shape: pl.BlockSpec(shape, lambda i: (0,) * len(shape))
    out_shapes = [
        jax.ShapeDtypeStruct((n, d), jnp.float32),
        jax.ShapeDtypeStruct((nt, n, d), jnp.bfloat16),
    ]
    out_specs = [
        pl.BlockSpec((bm, d), lambda i: (i, 0)),
        pl.BlockSpec((nt, bm, d), lambda i: (0, i, 0)),
    ]
    if write_bf16:
        out_shapes.append(jax.ShapeDtypeStruct((nt, n, n), jnp.bfloat16))
        out_specs.append(pl.BlockSpec((nt, bm, n), lambda i: (0, i, 0)))

    return pl.pallas_call(
        functools.partial(_step_kernel, t_fwd=t_fwd, write_bf16=write_bf16),
        grid=(nb,),
        in_specs=[
            pl.BlockSpec((nt, bm, n), lambda i: (0, i, 0)),
            full((nt, n, d)),
            pl.BlockSpec((bm, d), lambda i: (i, 0)),
            full((3 * d, d)),
            full((3 * d, d)),
            full((3 * d, d)),
            full((1, d)),
            full((1, d)),
            full((1, d)),
            full((d, nt * d)),
            full((1, nt * d)),
        ],
        out_specs=out_specs,
        out_shape=out_shapes,
        compiler_params=pltpu.CompilerParams(
            dimension_semantics=("arbitrary",),
        ),
    )(adj_tensor, msgs, h, Wz, Wr, Wh,
      bz.reshape(1, d), br.reshape(1, d), bh.reshape(1, d), We_flat, be_flat)


def kernel(adj_tensor, node_labels, n_prop_steps, emb, W_edge, b_edge,
           Wz, bz, Wr, br, Wh, bh):
    nt, _, d = W_edge.shape
    We_flat = W_edge.transpose(1, 0, 2).reshape(d, nt * d)
    be_flat = b_edge.reshape(1, nt * d)

    h0, msgs0 = _gather_and_msgs(node_labels, emb, We_flat, be_flat, nt)

    h1, msgs1, adj16 = _prop_step(adj_tensor, msgs0, h0, Wz, bz, Wr, br,
                                  Wh, bh, We_flat, be_flat,
                                  bm=256, write_bf16=True)

    def body(_, carry):
        h, msgs = carry
        h2, msgs2 = _prop_step(adj16, msgs, h, Wz, bz, Wr, br, Wh, bh,
                               We_flat, be_flat, bm=128, write_bf16=False)
        return (h2, msgs2)

    h_fin, _ = jax.lax.fori_loop(0, n_prop_steps - 1, body, (h1, msgs1))
    return h_fin


# speedup vs baseline: 1.0902x; 1.0902x over previous
import functools

import jax
import jax.numpy as jnp
from jax.experimental import pallas as pl
from jax.experimental.pallas import tpu as pltpu


def _msgs_from_h(h_blk, wef_ref, bef_ref, msgs_out_ref, nt, d):
    m = (jnp.dot(h_blk, wef_ref[:], preferred_element_type=jnp.float32)
         + bef_ref[:])
    for t in range(nt):
        msgs_out_ref[t] = m[:, t * d:(t + 1) * d].astype(jnp.bfloat16)


def _gather_kernel(labels_ref, emb_ref, wef_ref, bef_ref,
                   h_ref, msgs_ref, *, nt, d):
    n = labels_ref.shape[0]
    lpad = emb_ref.shape[0]
    lab = labels_ref[:]
    iota = jax.lax.broadcasted_iota(jnp.int32, (n, lpad), 1)
    onehot = (lab == iota).astype(jnp.float32)
    h0 = jnp.dot(onehot, emb_ref[:], preferred_element_type=jnp.float32)
    h_ref[:] = h0
    _msgs_from_h(h0, wef_ref, bef_ref, msgs_ref, nt, d)


def _gather_and_msgs(node_labels, emb, We_flat, be_flat, nt):
    n = node_labels.shape[0]
    l, d = emb.shape
    lpad = ((l + 127) // 128) * 128
    emb_p = jnp.pad(emb, ((0, lpad - l), (0, 0)))
    labels2d = node_labels.astype(jnp.int32).reshape(n, 1)
    return pl.pallas_call(
        functools.partial(_gather_kernel, nt=nt, d=d),
        out_shape=[
            jax.ShapeDtypeStruct((n, d), jnp.float32),
            jax.ShapeDtypeStruct((nt, n, d), jnp.bfloat16),
        ],
    )(labels2d, emb_p, We_flat, be_flat)


def _step_kernel(adj_ref, msgs_ref, h_ref, Wz_ref, Wr_ref, Wh_ref,
                 bz_ref, br_ref, bh_ref, wef_ref, bef_ref, *rest,
                 t_fwd, write_bf16):
    if write_bf16:
        out_ref, msgs_out_ref, adj16_ref = rest
    else:
        out_ref, msgs_out_ref = rest
        adj16_ref = None
    nt = msgs_ref.shape[0]
    d = msgs_ref.shape[2]

    slabs = []
    for t in range(nt):
        a = adj_ref[t]
        if a.dtype != jnp.bfloat16:
            a = a.astype(jnp.bfloat16)
        slabs.append(a)
        if write_bf16:
            adj16_ref[t] = a

    a_in = jnp.dot(slabs[0], msgs_ref[0], preferred_element_type=jnp.float32)
    for t in range(1, t_fwd):
        a_in += jnp.dot(slabs[t], msgs_ref[t], preferred_element_type=jnp.float32)
    a_out = jnp.dot(slabs[t_fwd], msgs_ref[t_fwd], preferred_element_type=jnp.float32)
    for t in range(t_fwd + 1, nt):
        a_out += jnp.dot(slabs[t], msgs_ref[t], preferred_element_type=jnp.float32)

    h_blk = h_ref[:]

    def lin3(a, b, c, w_ref, bias_ref):
        return (
            jnp.dot(a, w_ref[0:d, :], preferred_element_type=jnp.float32)
            + jnp.dot(b, w_ref[d:2 * d, :], preferred_element_type=jnp.float32)
            + jnp.dot(c, w_ref[2 * d:3 * d, :], preferred_element_type=jnp.float32)
            + bias_ref[:]
        )

    z = jax.nn.sigmoid(lin3(a_in, a_out, h_blk, Wz_ref, bz_ref))
    r = jax.nn.sigmoid(lin3(a_in, a_out, h_blk, Wr_ref, br_ref))
    h_hat = jnp.tanh(lin3(a_in, a_out, r * h_blk, Wh_ref, bh_ref))
    h_new = (1.0 - z) * h_blk + z * h_hat
    out_ref[:] = h_new
    _msgs_from_h(h_new, wef_ref, bef_ref, msgs_out_ref, nt, d)


def _prop_step(adj_tensor, msgs, h, Wz, bz, Wr, br, Wh, bh,
               We_flat, be_flat, *, bm, write_bf16):
    nt, n, _ = adj_tensor.shape
    d = h.shape[1]
    nb = n // bm
    t_fwd = nt // 2

    full = lambda shape: pl.BlockSpec(shape, lambda i: (0,) * len(shape))
    out_shapes = [
        jax.ShapeDtypeStruct((n, d), jnp.float32),
        jax.ShapeDtypeStruct((nt, n, d), jnp.bfloat16),
    ]
    out_specs = [
        pl.BlockSpec((bm, d), lambda i: (i, 0)),
        pl.BlockSpec((nt, bm, d), lambda i: (0, i, 0)),
    ]
    if write_bf16:
        out_shapes.append(jax.ShapeDtypeStruct((nt, n, n), jnp.bfloat16))
        out_specs.append(pl.BlockSpec((nt, bm, n), lambda i: (0, i, 0)))

    return pl.pallas_call(
        functools.partial(_step_kernel, t_fwd=t_fwd, write_bf16=write_bf16),
        grid=(nb,),
        in_specs=[
            pl.BlockSpec((nt, bm, n), lambda i: (0, i, 0)),
            full((nt, n, d)),
            pl.BlockSpec((bm, d), lambda i: (i, 0)),
            full((3 * d, d)),
            full((3 * d, d)),
            full((3 * d, d)),
            full((1, d)),
            full((1, d)),
            full((1, d)),
            full((d, nt * d)),
            full((1, nt * d)),
        ],
        out_specs=out_specs,
        out_shape=out_shapes,
        compiler_params=pltpu.CompilerParams(
            dimension_semantics=("arbitrary",),
        ),
    )(adj_tensor, msgs, h, Wz, Wr, Wh,
      bz.reshape(1, d), br.reshape(1, d), bh.reshape(1, d), We_flat, be_flat)


def kernel(adj_tensor, node_labels, n_prop_steps, emb, W_edge, b_edge,
           Wz, bz, Wr, br, Wh, bh):
    nt, _, d = W_edge.shape
    We_flat = W_edge.transpose(1, 0, 2).reshape(d, nt * d)
    be_flat = b_edge.reshape(1, nt * d)

    h0, msgs0 = _gather_and_msgs(node_labels, emb, We_flat, be_flat, nt)

    h1, msgs1, adj16 = _prop_step(adj_tensor, msgs0, h0, Wz, bz, Wr, br,
                                  Wh, bh, We_flat, be_flat,
                                  bm=256, write_bf16=True)

    def body(_, carry):
        h, msgs = carry
        h2, msgs2 = _prop_step(adj16, msgs, h, Wz, bz, Wr, br, Wh, bh,
                               We_flat, be_flat, bm=512, write_bf16=False)
        return (h2, msgs2)

    h_fin, _ = jax.lax.fori_loop(0, n_prop_steps - 1, body, (h1, msgs1))
    return h_fin


# speedup vs baseline: 1.1068x; 1.0152x over previous
import functools

import jax
import jax.numpy as jnp
from jax.experimental import pallas as pl
from jax.experimental.pallas import tpu as pltpu


def _mega_kernel(ns_ref, labels_ref, emb_ref, wef_ref, bef_ref,
                 Wz_ref, Wr_ref, Wh_ref, bz_ref, br_ref, bh_ref,
                 adj_ref, h_ref, adj16_ref,
                 inb, bfb, msgs_ref, in_sems, out_sems,
                 *, bm, nt, d, t_fwd):
    n = h_ref.shape[0]
    nb = n // bm

    def in_copy_f32(i, slot):
        return pltpu.make_async_copy(
            adj_ref.at[:, pl.ds(i * bm, bm), :], inb.at[slot],
            in_sems.at[slot])

    def out_copy(i, slot):
        return pltpu.make_async_copy(
            bfb.at[slot], adj16_ref.at[:, pl.ds(i * bm, bm), :],
            out_sems.at[slot])

    def in_copy_bf16(i, slot):
        return pltpu.make_async_copy(
            adj16_ref.at[:, pl.ds(i * bm, bm), :], bfb.at[slot],
            in_sems.at[slot])

    in_copy_f32(0, 0).start()

    lab = labels_ref[:]
    iota = jax.lax.broadcasted_iota(jnp.int32, (n, emb_ref.shape[0]), 1)
    onehot = (lab == iota).astype(jnp.float32)
    h_ref[:] = jnp.dot(onehot, emb_ref[:], preferred_element_type=jnp.float32)

    def refresh_msgs():
        m = (jnp.dot(h_ref[:], wef_ref[:], preferred_element_type=jnp.float32)
             + bef_ref[:])
        for t in range(nt):
            msgs_ref[t] = m[:, t * d:(t + 1) * d].astype(jnp.bfloat16)

    refresh_msgs()

    def lin3(a, b, c, w_ref, bias_ref):
        return (
            jnp.dot(a, w_ref[0:d, :], preferred_element_type=jnp.float32)
            + jnp.dot(b, w_ref[d:2 * d, :], preferred_element_type=jnp.float32)
            + jnp.dot(c, w_ref[2 * d:3 * d, :], preferred_element_type=jnp.float32)
            + bias_ref[:]
        )

    def aggregate(slab):
        a_in = jnp.dot(slab(0), msgs_ref[0], preferred_element_type=jnp.float32)
        for t in range(1, t_fwd):
            a_in += jnp.dot(slab(t), msgs_ref[t],
                            preferred_element_type=jnp.float32)
        a_out = jnp.dot(slab(t_fwd), msgs_ref[t_fwd],
                        preferred_element_type=jnp.float32)
        for t in range(t_fwd + 1, nt):
            a_out += jnp.dot(slab(t), msgs_ref[t],
                             preferred_element_type=jnp.float32)
        return a_in, a_out

    def gru_update(i, a_in, a_out):
        h_blk = h_ref[i * bm:(i + 1) * bm, :]
        z = jax.nn.sigmoid(lin3(a_in, a_out, h_blk, Wz_ref, bz_ref))
        r = jax.nn.sigmoid(lin3(a_in, a_out, h_blk, Wr_ref, br_ref))
        h_hat = jnp.tanh(lin3(a_in, a_out, r * h_blk, Wh_ref, bh_ref))
        h_ref[i * bm:(i + 1) * bm, :] = (1.0 - z) * h_blk + z * h_hat

    for i in range(nb):
        b = i % 2
        if i + 1 < nb:
            in_copy_f32(i + 1, 1 - b).start()
        in_copy_f32(i, b).wait()
        if i >= 2:
            out_copy(i - 2, b).wait()
        for t in range(nt):
            bfb[b, t] = inb[b, t].astype(jnp.bfloat16)
        out_copy(i, b).start()
        a_in, a_out = aggregate(lambda t: bfb[b, t])
        gru_update(i, a_in, a_out)
    out_copy(nb - 2, nb % 2).wait()
    out_copy(nb - 1, 1 - nb % 2).wait()
    refresh_msgs()

    def step_body(_, carry):
        in_copy_bf16(0, 0).start()
        for i in range(nb):
            b = i % 2
            if i + 1 < nb:
                in_copy_bf16(i + 1, 1 - b).start()
            in_copy_bf16(i, b).wait()
            a_in, a_out = aggregate(lambda t: bfb[b, t])
            gru_update(i, a_in, a_out)
        refresh_msgs()
        return carry

    jax.lax.fori_loop(0, ns_ref[0] - 1, step_body, 0)


def kernel(adj_tensor, node_labels, n_prop_steps, emb, W_edge, b_edge,
           Wz, bz, Wr, br, Wh, bh):
    nt, n, _ = adj_tensor.shape
    d = emb.shape[1]
    l = emb.shape[0]
    lpad = ((l + 127) // 128) * 128
    bm = 256
    t_fwd = nt // 2

    We_flat = W_edge.transpose(1, 0, 2).reshape(d, nt * d)
    be_flat = b_edge.reshape(1, nt * d)
    emb_p = jnp.pad(emb, ((0, lpad - l), (0, 0)))
    labels2d = node_labels.astype(jnp.int32).reshape(n, 1)
    ns = jnp.asarray(n_prop_steps, jnp.int32).reshape(1)

    vmem = lambda: pl.BlockSpec(memory_space=pltpu.VMEM)
    h, _ = pl.pallas_call(
        functools.partial(_mega_kernel, bm=bm, nt=nt, d=d, t_fwd=t_fwd),
        in_specs=[
            pl.BlockSpec(memory_space=pltpu.SMEM),
            vmem(),
            vmem(),
            vmem(),
            vmem(),
            vmem(), vmem(), vmem(),
            vmem(), vmem(), vmem(),
            pl.BlockSpec(memory_space=pl.ANY),
        ],
        out_specs=[
            vmem(),
            pl.BlockSpec(memory_space=pl.ANY),
        ],
        out_shape=[
            jax.ShapeDtypeStruct((n, d), jnp.float32),
            jax.ShapeDtypeStruct((nt, n, n), jnp.bfloat16),
        ],
        scratch_shapes=[
            pltpu.VMEM((2, nt, bm, n), jnp.float32),
            pltpu.VMEM((2, nt, bm, n), jnp.bfloat16),
            pltpu.VMEM((nt, n, d), jnp.bfloat16),
            pltpu.SemaphoreType.DMA((2,)),
            pltpu.SemaphoreType.DMA((2,)),
        ],
    )(ns, labels2d, emb_p, We_flat, be_flat, Wz, Wr, Wh,
      bz.reshape(1, d), br.reshape(1, d), bh.reshape(1, d), adj_tensor)
    return h
